# baseline (speedup 1.0000x reference)
import jax
import jax.numpy as jnp
from jax import lax
from jax.experimental import pallas as pl
from jax.experimental.pallas import tpu as pltpu
from jax.experimental.pallas import tpu_sc as plsc

_NC = 2
_NS = 16
_L = 16
_NW = _NC * _NS

_B = 16384
_D = 1000

_COLS_PER_W = _B // _NW
_NGRP = _COLS_PER_W // _L
_CHUNK_ROWS = 56
_NBUF = 4
_CHUNK_STARTS = list(range(0, _D, _CHUNK_ROWS))
_NCHUNK = len(_CHUNK_STARTS)
_UNROLL = 8


def _onehot_body(xs_hbm, zeros_hbm, out_hbm, xs_v, *rest):
    bufs = rest[:_NBUF]
    sems = rest[_NBUF:]

    wid = lax.axis_index("s") * _NC + lax.axis_index("c")
    base_col = wid * _COLS_PER_W

    zeros16 = jnp.zeros((_L,), jnp.float32)
    ones16 = jnp.ones((_L,), jnp.float32)
    lane = lax.iota(jnp.int32, _L)

    zcopies = [
        pltpu.async_copy(zeros_hbm, bufs[b], sems[b]) for b in range(_NBUF)
    ]

    pltpu.sync_copy(xs_hbm.at[pl.ds(base_col, _COLS_PER_W)], xs_v)

    def _sweep(buf, chunk, val):
        lo = _CHUNK_STARTS[chunk]
        hi = min(lo + _CHUNK_ROWS, _D)

        def _grp(i, carry):
            for u in range(_UNROLL):
                g = i * _UNROLL + u
                c = xs_v[pl.ds(g * _L, _L)]
                mask = (c >= lo) & (c < hi)
                plsc.store_scatter(buf, [c - lo, g * _L + lane], val, mask=mask)
            return carry

        lax.fori_loop(0, _NGRP // _UNROLL, _grp, 0)

    copies = [None] * _NCHUNK
    for k in range(_NCHUNK):
        buf = bufs[k % _NBUF]
        sem = sems[k % _NBUF]
        if k < _NBUF:
            zcopies[k].wait()
        else:
            copies[k - _NBUF].wait()
            _sweep(buf, k - _NBUF, zeros16)
        _sweep(buf, k, ones16)
        lo = _CHUNK_STARTS[k]
        rows = min(_CHUNK_ROWS, _D - lo)
        dst = out_hbm.at[pl.ds(lo, rows), pl.ds(base_col, _COLS_PER_W)]
        if rows == _CHUNK_ROWS:
            src = buf
        else:
            src = buf.at[pl.ds(0, rows), :]
        copies[k] = pltpu.async_copy(src, dst, sem)

    for k in range(_NCHUNK - _NBUF, _NCHUNK):
        copies[k].wait()


@jax.jit
def _onehot(xs):
    mesh = plsc.VectorSubcoreMesh(core_axis_name="c", subcore_axis_name="s")
    run = pl.kernel(
        _onehot_body,
        out_type=jax.ShapeDtypeStruct((_D, _B), jnp.float32),
        mesh=mesh,
        scratch_types=(
            [pltpu.VMEM((_COLS_PER_W,), jnp.int32)]
            + [pltpu.VMEM((_CHUNK_ROWS, _COLS_PER_W), jnp.float32)] * _NBUF
            + [pltpu.SemaphoreType.DMA] * _NBUF
        ),
        compiler_params=pltpu.CompilerParams(
            needs_layout_passes=False,
            use_tc_tiling_on_sc=True,
        ),
    )
    zeros_block = jnp.zeros((_CHUNK_ROWS, _COLS_PER_W), jnp.float32)
    return run(xs.astype(jnp.int32), zeros_block).T


def kernel(xs, matrix):
    del matrix
    return _onehot(xs)


# speedup vs baseline: 1.1127x; 1.1127x over previous
import jax
import jax.numpy as jnp
from jax import lax
from jax.experimental import pallas as pl
from jax.experimental.pallas import tpu as pltpu
from jax.experimental.pallas import tpu_sc as plsc

_NC = 2
_NS = 16
_L = 16
_NW = _NC * _NS

_B = 16384
_D = 1000

_COLS_PER_W = _B // _NW
_NGRP = _COLS_PER_W // _L
_CHUNK_ROWS = 64
_NBUF = 3
_CHUNK_STARTS = list(range(0, _D, _CHUNK_ROWS))
_NCHUNK = len(_CHUNK_STARTS)
_UNROLL = 8


def _onehot_body(xs_hbm, zeros_hbm, out_hbm, xs_v, *rest):
    bufs = rest[:_NBUF]
    sems = rest[_NBUF:]

    wid = lax.axis_index("s") * _NC + lax.axis_index("c")
    base_col = wid * _COLS_PER_W

    zeros16 = jnp.zeros((_L,), jnp.float32)
    ones16 = jnp.ones((_L,), jnp.float32)
    lane = lax.iota(jnp.int32, _L)

    zcopies = [
        pltpu.async_copy(zeros_hbm, bufs[b], sems[b]) for b in range(_NBUF)
    ]

    pltpu.sync_copy(xs_hbm.at[pl.ds(base_col, _COLS_PER_W)], xs_v)

    def _sweep(buf, chunk, val):
        lo = _CHUNK_STARTS[chunk]
        hi = min(lo + _CHUNK_ROWS, _D)

        def _grp(i, carry):
            for u in range(_UNROLL):
                g = i * _UNROLL + u
                c = xs_v[pl.ds(g * _L, _L)]
                mask = (c >= lo) & (c < hi)
                plsc.store_scatter(buf, [c - lo, g * _L + lane], val, mask=mask)
            return carry

        lax.fori_loop(0, _NGRP // _UNROLL, _grp, 0)

    copies = [None] * _NCHUNK
    for k in range(_NCHUNK):
        buf = bufs[k % _NBUF]
        sem = sems[k % _NBUF]
        if k < _NBUF:
            zcopies[k].wait()
        else:
            copies[k - _NBUF].wait()
            _sweep(buf, k - _NBUF, zeros16)
        _sweep(buf, k, ones16)
        lo = _CHUNK_STARTS[k]
        rows = min(_CHUNK_ROWS, _D - lo)
        dst = out_hbm.at[pl.ds(lo, rows), pl.ds(base_col, _COLS_PER_W)]
        if rows == _CHUNK_ROWS:
            src = buf
        else:
            src = buf.at[pl.ds(0, rows), :]
        copies[k] = pltpu.async_copy(src, dst, sem)

    for k in range(_NCHUNK - _NBUF, _NCHUNK):
        copies[k].wait()


@jax.jit
def _onehot(xs):
    mesh = plsc.VectorSubcoreMesh(core_axis_name="c", subcore_axis_name="s")
    run = pl.kernel(
        _onehot_body,
        out_type=jax.ShapeDtypeStruct((_D, _B), jnp.float32),
        mesh=mesh,
        scratch_types=(
            [pltpu.VMEM((_COLS_PER_W,), jnp.int32)]
            + [pltpu.VMEM((_CHUNK_ROWS, _COLS_PER_W), jnp.float32)] * _NBUF
            + [pltpu.SemaphoreType.DMA] * _NBUF
        ),
        compiler_params=pltpu.CompilerParams(
            needs_layout_passes=False,
            use_tc_tiling_on_sc=True,
        ),
    )
    zeros_block = jnp.zeros((_CHUNK_ROWS, _COLS_PER_W), jnp.float32)
    return run(xs.astype(jnp.int32), zeros_block).T


def kernel(xs, matrix):
    del matrix
    return _onehot(xs)
